# baseline (speedup 1.0000x reference)
import functools

import jax
import jax.numpy as jnp
from jax import lax
from jax.experimental import pallas as pl
from jax.experimental.pallas import tpu as pltpu
from jax.experimental.pallas import tpu_sc as plsc

NC = 1024
KS = 128
NEG = -3.0e38
L = 16

_SC_CORES = 2
_SC_SUBCORES = 16
_NW = _SC_CORES * _SC_SUBCORES
_ROWS_PER_W = NC // _NW
_NBATCH = _ROWS_PER_W // L


def _iota16():
    return lax.broadcasted_iota(jnp.int32, (L,), 0)


def _splat(x, dtype=jnp.int32):
    return jnp.full((L,), x, dtype=dtype)


def matmul_body(e_ref, w_ref, m_ref):
    m_ref[...] = jnp.dot(e_ref[...], w_ref[...],
                         preferred_element_type=jnp.float32)


def _max4(vals, idxs):
    a, b, c, d = vals
    ia, ib, ic, id_ = idxs
    t1 = b > a
    m1 = jnp.maximum(a, b)
    i1 = jnp.where(t1, ib, ia)
    t2 = d > c
    m2 = jnp.maximum(c, d)
    i2 = jnp.where(t2, id_, ic)
    t3 = m2 > m1
    return jnp.maximum(m1, m2), jnp.where(t3, i2, i1)


def _table_body(m_hbm, idt_hbm, idc_hbm, pos_hbm, *scr):
    wid = lax.axis_index("s") * _SC_CORES + lax.axis_index("c")
    iota = _iota16()
    batches = []
    for b in range(_NBATCH):
        leaf, g0v, g0i, g1v, g1i, g2v, g2i, g3v, g3i, cur, pos = (
            scr[b * 11:(b + 1) * 11])
        lvl = ((leaf, None), (g0v, g0i), (g1v, g1i), (g2v, g2i), (g3v, g3i))
        batches.append((wid * _ROWS_PER_W + b * L, lvl, cur, pos))

    for t0, lvl, cur, pos in batches:
        pltpu.sync_copy(m_hbm.at[pl.ds(t0, L), :], lvl[0][0])
        pltpu.sync_copy(idt_hbm, pos)
        pltpu.sync_copy(idc_hbm, cur)

    for h in range(1, 5):

        def build(n, _, h=h):
            for _, lvl, _, _ in batches:
                srcv, srci = lvl[h - 1]
                dstv, dsti = lvl[h]
                vals, idxs = [], []
                for e in range(4):
                    vals.append(plsc.load_gather(srcv, [iota, _splat(4 * n + e)]))
                    if h == 1:
                        idxs.append(_splat(4 * n + e))
                    else:
                        idxs.append(plsc.load_gather(srci, [iota, _splat(4 * n + e)]))
                nv, ni = _max4(vals, idxs)
                plsc.store_scatter(dstv, [iota, _splat(n)], nv)
                plsc.store_scatter(dsti, [iota, _splat(n)], ni)
            return 0

        lax.fori_loop(0, NC // (4 ** h), build, 0)

    def step(i, _):
        for _, lvl, cur, pos in batches:
            leaf = lvl[0][0]
            g3v, g3i = lvl[4]
            rv = [plsc.load_gather(g3v, [iota, _splat(e)]) for e in range(4)]
            ri = [plsc.load_gather(g3i, [iota, _splat(e)]) for e in range(4)]
            _, s = _max4(rv, ri)

            j = plsc.load_gather(pos, [iota, s])
            a = plsc.load_gather(cur, [iota, _splat(i)])
            plsc.store_scatter(cur, [iota, j], a, mask=j < _splat(KS))
            plsc.store_scatter(pos, [iota, a], j, mask=a != s)
            plsc.store_scatter(pos, [iota, s], _splat(i))

            plsc.store_scatter(leaf, [iota, s], _splat(NEG, jnp.float32))
            for h in range(1, 5):
                srcv, srci = lvl[h - 1]
                dstv, dsti = lvl[h]
                g = lax.shift_right_logical(s, 2 * h)
                base = 4 * g
                vals, idxs = [], []
                for e in range(4):
                    vals.append(plsc.load_gather(srcv, [iota, base + e]))
                    if h == 1:
                        idxs.append(base + e)
                    else:
                        idxs.append(plsc.load_gather(srci, [iota, base + e]))
                nv, ni = _max4(vals, idxs)
                plsc.store_scatter(dstv, [iota, g], nv)
                plsc.store_scatter(dsti, [iota, g], ni)
        return 0

    lax.fori_loop(0, KS, step, 0)

    for t0, lvl, cur, pos in batches:
        pltpu.sync_copy(pos, pos_hbm.at[pl.ds(t0, L), :])


_QPW = 8192 // _NW
_QCH = 64


def _answer_body(pos_hbm, x1_hbm, x2_hbm, y2_hbm,
                 x1v, x2v, idxv, rowbuf, outv, sem):
    wid = lax.axis_index("s") * _SC_CORES + lax.axis_index("c")
    iota = _iota16()
    base = wid * _QPW

    pltpu.sync_copy(x1_hbm.at[pl.ds(base, _QPW)], x1v)
    pltpu.sync_copy(x2_hbm.at[pl.ds(base, _QPW)], x2v)

    for ch in range(_QPW // _QCH):
        for g in range(_QCH // L):
            idxv[pl.ds(g * L, L)] = x1v[pl.ds(ch * _QCH + g * L, L)]
        pltpu.async_copy(pos_hbm.at[idxv], rowbuf, sem).wait()
        for g in range(_QCH // L):
            q0 = ch * _QCH + g * L
            c = x2v[pl.ds(q0, L)]
            vals = plsc.load_gather(rowbuf, [_splat(g * L) + iota, c])
            outv[pl.ds(q0, L)] = vals

    pltpu.sync_copy(outv, y2_hbm.at[pl.ds(base, _QPW)])


def _run(x1f, x2f, E, W):
    nc = E.shape[0]

    m = pl.pallas_call(
        matmul_body,
        out_shape=jax.ShapeDtypeStruct((nc, nc), jnp.float32),
    )(E, W)

    idt = jnp.tile(jnp.arange(nc, dtype=jnp.int32)[None, :], (L, 1))
    idc = jnp.tile(jnp.arange(KS, dtype=jnp.int32)[None, :], (L, 1))

    mesh = plsc.VectorSubcoreMesh(core_axis_name="c", subcore_axis_name="s")

    table = functools.partial(
        pl.kernel,
        out_type=jax.ShapeDtypeStruct((nc, nc), jnp.int32),
        mesh=mesh,
        compiler_params=pltpu.CompilerParams(use_tc_tiling_on_sc=False, needs_layout_passes=False),
        scratch_types=[
            pltpu.VMEM((L, nc), jnp.float32),
            pltpu.VMEM((L, nc // 4), jnp.float32),
            pltpu.VMEM((L, nc // 4), jnp.int32),
            pltpu.VMEM((L, nc // 16), jnp.float32),
            pltpu.VMEM((L, nc // 16), jnp.int32),
            pltpu.VMEM((L, nc // 64), jnp.float32),
            pltpu.VMEM((L, nc // 64), jnp.int32),
            pltpu.VMEM((L, nc // 256), jnp.float32),
            pltpu.VMEM((L, nc // 256), jnp.int32),
            pltpu.VMEM((L, KS), jnp.int32),
            pltpu.VMEM((L, nc), jnp.int32),
        ] * _NBATCH,
    )(_table_body)
    pos_tab = table(m, idt, idc)

    answer = functools.partial(
        pl.kernel,
        out_type=jax.ShapeDtypeStruct((8192,), jnp.int32),
        mesh=mesh,
        compiler_params=pltpu.CompilerParams(use_tc_tiling_on_sc=False, needs_layout_passes=False),
        scratch_types=[
            pltpu.VMEM((_QPW,), jnp.int32),
            pltpu.VMEM((_QPW,), jnp.int32),
            pltpu.VMEM((_QCH,), jnp.int32),
            pltpu.VMEM((_QCH, NC), jnp.int32),
            pltpu.VMEM((_QPW,), jnp.int32),
            pltpu.SemaphoreType.DMA,
        ],
    )(_answer_body)
    return answer(pos_tab, x1f, x2f)


@jax.jit
def kernel(x, E, W):
    n = x.shape[1]
    split = n - n // 2
    x1 = x[:, :split]
    x2 = x[:, split:]
    y2 = _run(x1.reshape(-1), x2.reshape(-1), E, W)
    return jnp.concatenate([x1, y2.reshape(x2.shape).astype(x1.dtype)],
                           axis=1)


# speedup vs baseline: 1.0832x; 1.0832x over previous
import functools

import jax
import jax.numpy as jnp
from jax import lax
from jax.experimental import pallas as pl
from jax.experimental.pallas import tpu as pltpu
from jax.experimental.pallas import tpu_sc as plsc

NC = 1024
KS = 128
NEG = -3.0e38
L = 16

_SC_CORES = 2
_SC_SUBCORES = 16
_NW = _SC_CORES * _SC_SUBCORES
_ROWS_PER_W = NC // _NW
_NBATCH = _ROWS_PER_W // L


def _iota16():
    return lax.broadcasted_iota(jnp.int32, (L,), 0)


def _splat(x, dtype=jnp.int32):
    return jnp.full((L,), x, dtype=dtype)


def matmul_body(e_ref, w_ref, m_ref):
    m_ref[...] = jnp.dot(e_ref[...], w_ref[...],
                         preferred_element_type=jnp.float32)


def _max4(vals, idxs):
    a, b, c, d = vals
    ia, ib, ic, id_ = idxs
    t1 = b > a
    m1 = jnp.maximum(a, b)
    i1 = jnp.where(t1, ib, ia)
    t2 = d > c
    m2 = jnp.maximum(c, d)
    i2 = jnp.where(t2, id_, ic)
    t3 = m2 > m1
    return jnp.maximum(m1, m2), jnp.where(t3, i2, i1)


def _table_body(m_hbm, idt_hbm, idc_hbm, pos_hbm, *scr):
    wid = lax.axis_index("s") * _SC_CORES + lax.axis_index("c")
    iota = _iota16()
    batches = []
    for b in range(_NBATCH):
        leaf, g0v, g0i, g1v, g1i, g2v, g2i, g3v, g3i, cur, pos = (
            scr[b * 11:(b + 1) * 11])
        lvl = ((leaf, None), (g0v, g0i), (g1v, g1i), (g2v, g2i), (g3v, g3i))
        batches.append((wid * _ROWS_PER_W + b * L, lvl, cur, pos))

    for t0, lvl, cur, pos in batches:
        pltpu.sync_copy(m_hbm.at[pl.ds(t0, L), :], lvl[0][0])
        pltpu.sync_copy(idt_hbm, pos)
        pltpu.sync_copy(idc_hbm, cur)

    for h in range(1, 5):

        def build(n, _, h=h):
            for _, lvl, _, _ in batches:
                srcv, srci = lvl[h - 1]
                dstv, dsti = lvl[h]
                vals, idxs = [], []
                for e in range(4):
                    vals.append(plsc.load_gather(srcv, [iota, _splat(4 * n + e)]))
                    if h == 1:
                        idxs.append(_splat(4 * n + e))
                    else:
                        idxs.append(plsc.load_gather(srci, [iota, _splat(4 * n + e)]))
                nv, ni = _max4(vals, idxs)
                plsc.store_scatter(dstv, [iota, _splat(n)], nv)
                plsc.store_scatter(dsti, [iota, _splat(n)], ni)
            return 0

        lax.fori_loop(0, NC // (4 ** h), build, 0)

    def step(i, _):
        for _, lvl, cur, pos in batches:
            leaf = lvl[0][0]
            g3v, g3i = lvl[4]
            rv = [plsc.load_gather(g3v, [iota, _splat(e)]) for e in range(4)]
            ri = [plsc.load_gather(g3i, [iota, _splat(e)]) for e in range(4)]
            _, s = _max4(rv, ri)

            j = plsc.load_gather(pos, [iota, s])
            a = plsc.load_gather(cur, [iota, _splat(i)])
            plsc.store_scatter(cur, [iota, j], a, mask=j < _splat(KS))
            plsc.store_scatter(pos, [iota, a], j, mask=a != s)
            plsc.store_scatter(pos, [iota, s], _splat(i))

            plsc.store_scatter(leaf, [iota, s], _splat(NEG, jnp.float32))
            for h in range(1, 5):
                srcv, srci = lvl[h - 1]
                dstv, dsti = lvl[h]
                g = lax.shift_right_logical(s, 2 * h)
                base = 4 * g
                vals, idxs = [], []
                for e in range(4):
                    vals.append(plsc.load_gather(srcv, [iota, base + e]))
                    if h == 1:
                        idxs.append(base + e)
                    else:
                        idxs.append(plsc.load_gather(srci, [iota, base + e]))
                nv, ni = _max4(vals, idxs)
                plsc.store_scatter(dstv, [iota, g], nv)
                plsc.store_scatter(dsti, [iota, g], ni)
        return 0

    lax.fori_loop(0, KS, step, 0)

    for t0, lvl, cur, pos in batches:
        pltpu.sync_copy(pos, pos_hbm.at[pl.ds(t0, L), :])


_QPW = 8192 // _NW
_QCH = 64


def _answer_body(posf_hbm, x1_hbm, x2_hbm, y2_hbm,
                 x1v, x2v, idx2, outv, sem):
    wid = lax.axis_index("s") * _SC_CORES + lax.axis_index("c")
    base = wid * _QPW

    pltpu.sync_copy(x1_hbm.at[pl.ds(base, _QPW)], x1v)
    pltpu.sync_copy(x2_hbm.at[pl.ds(base, _QPW)], x2v)

    for k in range(_QPW // L):
        a = x1v[pl.ds(k * L, L)]
        c = x2v[pl.ds(k * L, L)]
        idx2[k * L // _QCH, pl.ds((k * L) % _QCH, L)] = a * NC + c

    for j in range(_QPW // _QCH):
        pltpu.async_copy(posf_hbm.at[idx2.at[j]],
                         outv.at[pl.ds(j * _QCH, _QCH)], sem).wait()

    pltpu.sync_copy(outv, y2_hbm.at[pl.ds(base, _QPW)])


def _run(x1f, x2f, E, W):
    nc = E.shape[0]

    m = pl.pallas_call(
        matmul_body,
        grid=(8,),
        in_specs=[
            pl.BlockSpec((nc // 8, nc), lambda i: (i, 0)),
            pl.BlockSpec((nc, nc), lambda i: (0, 0)),
        ],
        out_specs=pl.BlockSpec((nc // 8, nc), lambda i: (i, 0)),
        out_shape=jax.ShapeDtypeStruct((nc, nc), jnp.float32),
    )(E, W)

    idt = jnp.tile(jnp.arange(nc, dtype=jnp.int32)[None, :], (L, 1))
    idc = jnp.tile(jnp.arange(KS, dtype=jnp.int32)[None, :], (L, 1))

    mesh = plsc.VectorSubcoreMesh(core_axis_name="c", subcore_axis_name="s")

    table = functools.partial(
        pl.kernel,
        out_type=jax.ShapeDtypeStruct((nc, nc), jnp.int32),
        mesh=mesh,
        compiler_params=pltpu.CompilerParams(use_tc_tiling_on_sc=False, needs_layout_passes=False),
        scratch_types=[
            pltpu.VMEM((L, nc), jnp.float32),
            pltpu.VMEM((L, nc // 4), jnp.float32),
            pltpu.VMEM((L, nc // 4), jnp.int32),
            pltpu.VMEM((L, nc // 16), jnp.float32),
            pltpu.VMEM((L, nc // 16), jnp.int32),
            pltpu.VMEM((L, nc // 64), jnp.float32),
            pltpu.VMEM((L, nc // 64), jnp.int32),
            pltpu.VMEM((L, nc // 256), jnp.float32),
            pltpu.VMEM((L, nc // 256), jnp.int32),
            pltpu.VMEM((L, KS), jnp.int32),
            pltpu.VMEM((L, nc), jnp.int32),
        ] * _NBATCH,
    )(_table_body)
    pos_tab = table(m, idt, idc)

    answer = functools.partial(
        pl.kernel,
        out_type=jax.ShapeDtypeStruct((8192,), jnp.int32),
        mesh=mesh,
        compiler_params=pltpu.CompilerParams(use_tc_tiling_on_sc=False, needs_layout_passes=False),
        scratch_types=[
            pltpu.VMEM((_QPW,), jnp.int32),
            pltpu.VMEM((_QPW,), jnp.int32),
            pltpu.VMEM((_QPW // _QCH, _QCH), jnp.int32),
            pltpu.VMEM((_QPW,), jnp.int32),
            pltpu.SemaphoreType.DMA,
        ],
    )(_answer_body)
    return answer(pos_tab.reshape(-1), x1f, x2f)


@jax.jit
def kernel(x, E, W):
    n = x.shape[1]
    split = n - n // 2
    x1 = x[:, :split]
    x2 = x[:, split:]
    y2 = _run(x1.reshape(-1), x2.reshape(-1), E, W)
    return jnp.concatenate([x1, y2.reshape(x2.shape).astype(x1.dtype)],
                           axis=1)
